# baseline (speedup 1.0000x reference)
import functools

import jax
import jax.numpy as jnp
from jax import lax
from jax.experimental import pallas as pl
from jax.experimental.pallas import tpu as pltpu
from jax.experimental.pallas import tpu_sc as plsc

_D = 64
_DP = 128
_B = 16384

_info = plsc.get_sparse_core_info()
_NC, _NS, _L = _info.num_cores, _info.num_subcores, _info.num_lanes
_NW = _NC * _NS
_KU = 262144
_KM = 32768
_BPW = _B // _NW
_C = 64
_NCHUNK = _BPW // _C
_GRP = _C // _L


def _padt_body(q0_ref, q1_ref, q2_ref, q3_ref, out_ref):
    def trp(x):
        p = pltpu.bitcast(x.astype(jnp.bfloat16), jnp.int32)
        return jnp.transpose(p)

    h = _D // 2
    out_ref[:, 0 * h:1 * h] = trp(q0_ref[...])
    out_ref[:, 1 * h:2 * h] = trp(q1_ref[...])
    out_ref[:, 2 * h:3 * h] = trp(q2_ref[...])
    out_ref[:, 3 * h:4 * h] = trp(q3_ref[...])


def _padt_table(table_t, k, block_cols):
    nb = k // block_cols
    nlast = pl.cdiv(table_t.shape[1], block_cols) - 1

    def qspec(q):
        return pl.BlockSpec(
            (_D, block_cols),
            lambda i, _q=q, _nb=nb, _nl=nlast: (0, jnp.minimum(i + _q * _nb, _nl)))

    return pl.pallas_call(
        _padt_body,
        grid=(nb,),
        in_specs=[qspec(0), qspec(1), qspec(2), qspec(3)],
        out_specs=pl.BlockSpec((block_cols, _DP), lambda i: (i, 0)),
        out_shape=jax.ShapeDtypeStruct((k, _DP), jnp.int32),
        compiler_params=pltpu.CompilerParams(
            dimension_semantics=("arbitrary",),
            vmem_limit_bytes=55 * 1024 * 1024),
    )(table_t, table_t, table_t, table_t)


def _bias_body(in_ref, out_ref):
    out_ref[...] = in_ref[...].reshape(out_ref.shape)


def _bias_lin(bias_t, block_cols):
    n = bias_t.shape[1]
    return pl.pallas_call(
        _bias_body,
        grid=(pl.cdiv(n, block_cols),),
        in_specs=[pl.BlockSpec((1, block_cols), lambda i: (0, i))],
        out_specs=pl.BlockSpec((block_cols,), lambda i: (i,)),
        out_shape=jax.ShapeDtypeStruct((n,), jnp.float32),
        compiler_params=pltpu.CompilerParams(
            dimension_semantics=("arbitrary",)),
    )(bias_t)


def _mf_kernel(user_hbm, mission_hbm, utab_hbm, mtab_hbm, ubias_hbm,
               mbias_hbm, out_hbm,
               uidx_v, midx_v, ublk_v, mblk_v, ubufA, mbufA, ubufB, mbufB,
               biasu_v, biasm_v, outv, semA, semB, semC):
    wid = lax.axis_index("s") * _NC + lax.axis_index("c")
    base = wid * _BPW

    pltpu.sync_copy(user_hbm.at[pl.ds(base, _BPW)], uidx_v)
    pltpu.sync_copy(mission_hbm.at[pl.ds(base, _BPW)], midx_v)

    bias_cps = [
        pltpu.async_copy(ubias_hbm.at[uidx_v], biasu_v, semC),
        pltpu.async_copy(mbias_hbm.at[midx_v], biasm_v, semC),
    ]

    def blk_body(i, carry):
        sl = pl.ds(i * _L, _L)
        u = uidx_v[sl]
        m = midx_v[sl]
        ublk_v[sl] = lax.bitwise_and(u, _KU - 1)
        mblk_v[sl] = lax.bitwise_and(m, _KM - 1)
        return carry

    lax.fori_loop(0, _BPW // _L, blk_body, 0)

    def start(c, ubuf, mbuf, sem):
        sl = pl.ds(c * _C, _C)
        pltpu.async_copy(utab_hbm.at[ublk_v.at[sl]], ubuf, sem)
        pltpu.async_copy(mtab_hbm.at[mblk_v.at[sl]], mbuf, sem)

    def wait(ubuf, mbuf, sem):
        pltpu.make_async_copy(utab_hbm.at[pl.ds(0, _C)], ubuf, sem).wait()
        pltpu.make_async_copy(mtab_hbm.at[pl.ds(0, _C)], mbuf, sem).wait()

    lane = jnp.arange(_L, dtype=jnp.int32)

    def compute(c, ubuf, mbuf):
        def grp_body(g, carry):
            sl = pl.ds(c * _C + g * _L, _L)
            acc = biasu_v[sl] + biasm_v[sl]
            row = g * _L + lane
            h = _D // 2
            ucol0 = lax.shift_right_logical(uidx_v[sl], 18) * h
            mcol0 = lax.shift_right_logical(midx_v[sl], 15) * h
            for d2 in range(h):
                up = plsc.load_gather(ubuf, [row, ucol0 + d2])
                mp = plsc.load_gather(mbuf, [row, mcol0 + d2])
                ue, uo = plsc.unpack(plsc.bitcast(up, jnp.bfloat16),
                                     format=plsc.PackFormat.INTERLEAVED)
                me, mo = plsc.unpack(plsc.bitcast(mp, jnp.bfloat16),
                                     format=plsc.PackFormat.INTERLEAVED)
                acc = acc + ue * me + uo * mo
            outv[sl] = acc
            return carry

        lax.fori_loop(0, _GRP, grp_body, 0)

    start(0, ubufA, mbufA, semA)
    for cp in bias_cps:
        cp.wait()

    def pipe_body(g, carry):
        cA = 2 * g
        cB = cA + 1
        start(cB, ubufB, mbufB, semB)
        wait(ubufA, mbufA, semA)
        compute(cA, ubufA, mbufA)

        @pl.when(cA + 2 < _NCHUNK)
        def _():
            start(cA + 2, ubufA, mbufA, semA)

        wait(ubufB, mbufB, semB)
        compute(cB, ubufB, mbufB)
        return carry

    lax.fori_loop(0, _NCHUNK // 2, pipe_body, 0)

    pltpu.sync_copy(outv, out_hbm.at[pl.ds(base, _BPW)])


@jax.jit
def _mf(user, mission, user_table, mission_table, user_bias, mission_bias):
    utab = _padt_table(user_table.T, _KU, 8192)
    mtab = _padt_table(mission_table.T, _KM, 8192)
    user_bias = _bias_lin(user_bias.T, 65536)
    mission_bias = _bias_lin(mission_bias.T, 65536)
    mesh = plsc.VectorSubcoreMesh(core_axis_name="c", subcore_axis_name="s")
    run = functools.partial(
        pl.kernel,
        mesh=mesh,
        compiler_params=pltpu.CompilerParams(
            needs_layout_passes=False, use_tc_tiling_on_sc=False),
        out_type=jax.ShapeDtypeStruct((_B,), jnp.float32),
        scratch_types=[
            pltpu.VMEM((_BPW,), jnp.int32),
            pltpu.VMEM((_BPW,), jnp.int32),
            pltpu.VMEM((_BPW,), jnp.int32),
            pltpu.VMEM((_BPW,), jnp.int32),
            pltpu.VMEM((_C, _DP), jnp.int32),
            pltpu.VMEM((_C, _DP), jnp.int32),
            pltpu.VMEM((_C, _DP), jnp.int32),
            pltpu.VMEM((_C, _DP), jnp.int32),
            pltpu.VMEM((_BPW,), jnp.float32),
            pltpu.VMEM((_BPW,), jnp.float32),
            pltpu.VMEM((_BPW,), jnp.float32),
            pltpu.SemaphoreType.DMA,
            pltpu.SemaphoreType.DMA,
            pltpu.SemaphoreType.DMA,
        ],
    )(_mf_kernel)
    return run(user, mission, utab, mtab, user_bias, mission_bias)


def kernel(user, mission, user_table, mission_table, user_bias, mission_bias):
    return _mf(user.astype(jnp.int32), mission.astype(jnp.int32),
               user_table, mission_table, user_bias, mission_bias)


# speedup vs baseline: 1.5446x; 1.5446x over previous
import functools

import jax
import jax.numpy as jnp
from jax import lax
from jax.experimental import pallas as pl
from jax.experimental.pallas import tpu as pltpu
from jax.experimental.pallas import tpu_sc as plsc

_D = 64
_DP = 128
_B = 16384

_info = plsc.get_sparse_core_info()
_NC, _NS, _L = _info.num_cores, _info.num_subcores, _info.num_lanes
_NW = _NC * _NS
_KU = 524288
_KM = 65536
_BPW = _B // _NW
_C = 64
_NCHUNK = _BPW // _C
_GRP = _C // _L


def _padt_body(lo_ref, hi_ref, out_ref):
    ident = jnp.eye(_D, dtype=jnp.bfloat16)

    def trp(x):
        xt = jax.lax.dot_general(
            x.astype(jnp.bfloat16), ident, (((0,), (0,)), ((), ())),
            preferred_element_type=jnp.float32)
        return pltpu.bitcast(xt.astype(jnp.bfloat16), jnp.int32)

    out_ref[:, 0:_D] = trp(lo_ref[...])
    out_ref[:, _D:_DP] = trp(hi_ref[...])


def _padt_table(table_t, k, block_cols):
    nb = k // block_cols
    nlast = pl.cdiv(table_t.shape[1], block_cols) - 1
    return pl.pallas_call(
        _padt_body,
        grid=(nb,),
        in_specs=[pl.BlockSpec((_D, block_cols), lambda i: (0, i)),
                  pl.BlockSpec(
                      (_D, block_cols),
                      lambda i, _nb=nb, _nl=nlast: (0, jnp.minimum(i + _nb, _nl)))],
        out_specs=pl.BlockSpec((block_cols // 2, _DP), lambda i: (i, 0)),
        out_shape=jax.ShapeDtypeStruct((k // 2, _DP), jnp.int32),
        compiler_params=pltpu.CompilerParams(
            dimension_semantics=("arbitrary",),
            vmem_limit_bytes=55 * 1024 * 1024),
    )(table_t, table_t)


def _bias_body(in_ref, out_ref):
    out_ref[...] = in_ref[...].reshape(out_ref.shape)


def _bias_lin(bias_t, block_cols):
    n = bias_t.shape[1]
    return pl.pallas_call(
        _bias_body,
        grid=(pl.cdiv(n, block_cols),),
        in_specs=[pl.BlockSpec((1, block_cols), lambda i: (0, i))],
        out_specs=pl.BlockSpec((block_cols,), lambda i: (i,)),
        out_shape=jax.ShapeDtypeStruct((n,), jnp.float32),
        compiler_params=pltpu.CompilerParams(
            dimension_semantics=("arbitrary",)),
    )(bias_t)


def _mf_kernel(user_hbm, mission_hbm, utab_hbm, mtab_hbm, ubias_hbm,
               mbias_hbm, out_hbm,
               uidx_v, midx_v, ublk_v, mblk_v, ubufA, mbufA, ubufB, mbufB,
               biasu_v, biasm_v, outv, semA, semB, semC):
    wid = lax.axis_index("s") * _NC + lax.axis_index("c")
    base = wid * _BPW

    pltpu.sync_copy(user_hbm.at[pl.ds(base, _BPW)], uidx_v)
    pltpu.sync_copy(mission_hbm.at[pl.ds(base, _BPW)], midx_v)

    bias_cps = [
        pltpu.async_copy(ubias_hbm.at[uidx_v], biasu_v, semC),
        pltpu.async_copy(mbias_hbm.at[midx_v], biasm_v, semC),
    ]

    def blk_body(i, carry):
        sl = pl.ds(i * _L, _L)
        u = uidx_v[sl]
        m = midx_v[sl]
        ublk_v[sl] = lax.shift_right_logical(lax.bitwise_and(u, _KU - 1), 1)
        mblk_v[sl] = lax.shift_right_logical(lax.bitwise_and(m, _KM - 1), 1)
        return carry

    lax.fori_loop(0, _BPW // _L, blk_body, 0)

    def start(c, ubuf, mbuf, sem):
        sl = pl.ds(c * _C, _C)
        pltpu.async_copy(utab_hbm.at[ublk_v.at[sl]], ubuf, sem)
        pltpu.async_copy(mtab_hbm.at[mblk_v.at[sl]], mbuf, sem)

    def wait(ubuf, mbuf, sem):
        pltpu.make_async_copy(utab_hbm.at[pl.ds(0, _C)], ubuf, sem).wait()
        pltpu.make_async_copy(mtab_hbm.at[pl.ds(0, _C)], mbuf, sem).wait()

    lane = jnp.arange(_L, dtype=jnp.int32)

    def compute(c, ubuf, mbuf):
        def grp_body(g, carry):
            sl = pl.ds(c * _C + g * _L, _L)
            acc = biasu_v[sl] + biasm_v[sl]
            row = g * _L + lane
            u = uidx_v[sl]
            m = midx_v[sl]
            ucol0 = lax.shift_right_logical(u, 19) * _D
            mcol0 = lax.shift_right_logical(m, 16) * _D
            usel = lax.bitwise_and(u, 1) == 1
            msel = lax.bitwise_and(m, 1) == 1
            for d in range(_D):
                up = plsc.load_gather(ubuf, [row, ucol0 + d])
                mp = plsc.load_gather(mbuf, [row, mcol0 + d])
                ua, ub = plsc.unpack(plsc.bitcast(up, jnp.bfloat16),
                                     format=plsc.PackFormat.INTERLEAVED)
                ma, mb2 = plsc.unpack(plsc.bitcast(mp, jnp.bfloat16),
                                      format=plsc.PackFormat.INTERLEAVED)
                uu = jnp.where(usel, ub, ua)
                mm = jnp.where(msel, mb2, ma)
                acc = acc + uu * mm
            outv[sl] = acc
            return carry

        lax.fori_loop(0, _GRP, grp_body, 0)

    start(0, ubufA, mbufA, semA)
    for cp in bias_cps:
        cp.wait()

    def pipe_body(g, carry):
        cA = 2 * g
        cB = cA + 1
        start(cB, ubufB, mbufB, semB)
        wait(ubufA, mbufA, semA)
        compute(cA, ubufA, mbufA)

        @pl.when(cA + 2 < _NCHUNK)
        def _():
            start(cA + 2, ubufA, mbufA, semA)

        wait(ubufB, mbufB, semB)
        compute(cB, ubufB, mbufB)
        return carry

    lax.fori_loop(0, _NCHUNK // 2, pipe_body, 0)

    pltpu.sync_copy(outv, out_hbm.at[pl.ds(base, _BPW)])


@jax.jit
def _mf(user, mission, user_table, mission_table, user_bias, mission_bias):
    utab = _padt_table(user_table.T, _KU, 16384)
    mtab = _padt_table(mission_table.T, _KM, 16384)
    user_bias = _bias_lin(user_bias.T, 65536)
    mission_bias = _bias_lin(mission_bias.T, 65536)
    mesh = plsc.VectorSubcoreMesh(core_axis_name="c", subcore_axis_name="s")
    run = functools.partial(
        pl.kernel,
        mesh=mesh,
        compiler_params=pltpu.CompilerParams(
            needs_layout_passes=False, use_tc_tiling_on_sc=False),
        out_type=jax.ShapeDtypeStruct((_B,), jnp.float32),
        scratch_types=[
            pltpu.VMEM((_BPW,), jnp.int32),
            pltpu.VMEM((_BPW,), jnp.int32),
            pltpu.VMEM((_BPW,), jnp.int32),
            pltpu.VMEM((_BPW,), jnp.int32),
            pltpu.VMEM((_C, _DP), jnp.int32),
            pltpu.VMEM((_C, _DP), jnp.int32),
            pltpu.VMEM((_C, _DP), jnp.int32),
            pltpu.VMEM((_C, _DP), jnp.int32),
            pltpu.VMEM((_BPW,), jnp.float32),
            pltpu.VMEM((_BPW,), jnp.float32),
            pltpu.VMEM((_BPW,), jnp.float32),
            pltpu.SemaphoreType.DMA,
            pltpu.SemaphoreType.DMA,
            pltpu.SemaphoreType.DMA,
        ],
    )(_mf_kernel)
    return run(user, mission, utab, mtab, user_bias, mission_bias)


def kernel(user, mission, user_table, mission_table, user_bias, mission_bias):
    return _mf(user.astype(jnp.int32), mission.astype(jnp.int32),
               user_table, mission_table, user_bias, mission_bias)
